# baseline (speedup 1.0000x reference)
import functools

import jax
import jax.numpy as jnp
from jax import lax
from jax.experimental import pallas as pl
from jax.experimental.pallas import tpu as pltpu
from jax.experimental.pallas import tpu_sc as plsc

EMB = 64
NC = 2
NS = 16
NW = NC * NS
CHUNK = 80
HALF = CHUNK // 2
GRP = 8


@functools.lru_cache(maxsize=None)
def _build(B, L):
    N = B * L
    n_per_w = N // NW
    n_chunks = n_per_w // CHUNK
    n_groups = n_chunks // GRP
    p_per_w = n_per_w // 2
    mesh = plsc.VectorSubcoreMesh(core_axis_name="c", subcore_axis_name="s")

    @functools.partial(
        pl.kernel,
        mesh=mesh,
        compiler_params=pltpu.CompilerParams(use_tc_tiling_on_sc=False),
        out_type=jax.ShapeDtypeStruct((N // 2, 2 * EMB), jnp.float32),
        scratch_types=[
            pltpu.VMEM((n_chunks, CHUNK), jnp.int32),
            pltpu.VMEM((n_chunks, CHUNK), jnp.int32),
            pltpu.VMEM((GRP, CHUNK, EMB), jnp.float32),
            pltpu.SemaphoreType.DMA,
            pltpu.SemaphoreType.DMA,
        ],
    )
    def emb_kernel(labels_hbm, l2g_hbm, table_hbm, x_hbm,
                   idx_v, gidx_v, rows_v, gsem, osem):
        wid = lax.axis_index("s") * NC + lax.axis_index("c")
        p_base = wid * p_per_w

        pltpu.sync_copy(labels_hbm.at[wid], idx_v)

        def gidx_body(jj, carry):
            handles = []
            for b in range(GRP):
                j = jj * GRP + b
                handles.append(
                    pltpu.async_copy(l2g_hbm.at[idx_v.at[j]], gidx_v.at[j],
                                     gsem))
            for h in handles:
                h.wait()
            return carry

        lax.fori_loop(0, n_groups, gidx_body, 0, unroll=False)

        def row_body(jj, carry):
            @pl.when(jj > 0)
            def _drain_prev():
                for b in range(GRP):
                    for h in range(2):
                        pltpu.make_async_copy(
                            rows_v.at[b, pl.ds(h * HALF, HALF)],
                            x_hbm.at[pl.ds(p_base, HALF),
                                     pl.ds(h * EMB, EMB)],
                            osem).wait()

            handles = []
            for b in range(GRP):
                j = jj * GRP + b
                handles.append(
                    pltpu.async_copy(table_hbm.at[gidx_v.at[j]], rows_v.at[b],
                                     gsem))
            for b in range(GRP):
                j = jj * GRP + b
                handles[b].wait()
                p0 = p_base + j * HALF
                for h in range(2):
                    pltpu.async_copy(
                        rows_v.at[b, pl.ds(h * HALF, HALF)],
                        x_hbm.at[pl.ds(p0, HALF), pl.ds(h * EMB, EMB)],
                        osem)
            return carry

        lax.fori_loop(0, n_groups, row_body, 0, unroll=False)
        for b in range(GRP):
            for h in range(2):
                pltpu.make_async_copy(
                    rows_v.at[b, pl.ds(h * HALF, HALF)],
                    x_hbm.at[pl.ds(p_base, HALF), pl.ds(h * EMB, EMB)],
                    osem).wait()

    return emb_kernel


def kernel(label_ids, local2global, table):
    B, L = label_ids.shape
    N = B * L
    labels = (label_ids.reshape(-1, HALF, 2)
              .transpose(0, 2, 1)
              .reshape(NW, N // NW // CHUNK, CHUNK))
    x = _build(B, L)(labels, local2global, table)
    return x.reshape(B, L, EMB)
